# Initial kernel scaffold; baseline (speedup 1.0000x reference)
import jax
import jax.numpy as jnp
from jax.experimental import pallas as pl


def kernel(node_states, edge_indices, Wq, bq, Wk, bk, Wv, bv):
    raise NotImplementedError("write your pallas kernel here")


# speedup vs baseline: 5.5125x; 5.5125x over previous
import functools
import math

import jax
import jax.numpy as jnp
from jax import lax
from jax.experimental import pallas as pl
from jax.experimental.pallas import tpu as pltpu
from jax.experimental.pallas import tpu_sc as plsc

H = 4
D = 128
DH = D // H
NC = 2
NS = 16
LANES = 16
CHUNK = 80
DEN_W = 16


def _qkv_body(x_ref, wq_ref, wk_ref, wv_ref, b_ref, q_ref, k_ref, v_ref):
    x = x_ref[...]
    scale = 1.0 / math.sqrt(DH)
    q = jnp.dot(x, wq_ref[...], preferred_element_type=jnp.float32)
    q_ref[...] = (q + b_ref[0:1, :]) * scale
    k_ref[...] = jnp.dot(x, wk_ref[...], preferred_element_type=jnp.float32) + b_ref[1:2, :]
    v_ref[...] = jnp.dot(x, wv_ref[...], preferred_element_type=jnp.float32) + b_ref[2:3, :]


def _qkv(x, wq, wk, wv, bpad):
    n = x.shape[0]
    return pl.pallas_call(
        _qkv_body,
        out_shape=[
            jax.ShapeDtypeStruct((n, D), jnp.float32),
            jax.ShapeDtypeStruct((n, D), jnp.float32),
            jax.ShapeDtypeStruct((n, D), jnp.float32),
        ],
    )(x, wq, wk, wv, bpad)


def _edge_body(npad, ept, q_hbm, k_hbm, v_hbm, dst_hbm, src_hbm,
               num_hbm, den_hbm,
               idx_d, idx_s, idx_d8, qnum, kbuf, ex_v, num_sh, den_sh,
               sem_q, sem_kv):
    c = lax.axis_index("c")
    s = lax.axis_index("s")
    rpt = npad // NS
    nchunk = ept // CHUNK

    z16 = jnp.zeros((LANES,), jnp.float32)

    def _zero_body(r, _):
        for k in range(D // LANES):
            qnum[r, k * LANES:(k + 1) * LANES] = z16
            ex_v[r, k * LANES:(k + 1) * LANES] = z16
        return 0
    lax.fori_loop(0, CHUNK, _zero_body, 0)

    r0 = s * rpt
    nfull = rpt // CHUNK
    rem = rpt - nfull * CHUNK
    for i in range(nfull):
        pltpu.sync_copy(qnum, num_sh.at[pl.ds(r0 + i * CHUNK, CHUNK)])
    if rem:
        pltpu.sync_copy(qnum.at[pl.ds(0, rem)],
                        num_sh.at[pl.ds(r0 + nfull * CHUNK, rem)])
    pltpu.sync_copy(ex_v, den_sh.at[pl.ds(s * (rpt // 8), rpt // 8)])
    plsc.subcore_barrier()

    base = (c * NS + s) * ept

    def _chunk(i, _):
        off = base + i * CHUNK
        pltpu.sync_copy(dst_hbm.at[pl.ds(off, CHUNK)], idx_d)
        pltpu.sync_copy(src_hbm.at[pl.ds(off, CHUNK)], idx_s)
        cq = pltpu.async_copy(q_hbm.at[idx_d], qnum, sem_q)
        ck = pltpu.async_copy(k_hbm.at[idx_s], kbuf, sem_kv)
        cq.wait()
        ck.wait()
        for g in range(CHUNK // LANES):
            rows = jnp.arange(LANES, dtype=jnp.int32) + (g * LANES)
            dvec = idx_d[g * LANES:(g + 1) * LANES]
            idx_d8[g * LANES:(g + 1) * LANES] = dvec >> 3
            cbase = (dvec & 7) << 4
            for h in range(H):
                def _dot(j, acc, h=h, rows=rows):
                    col = jnp.full((LANES,), h * DH, jnp.int32) + j
                    qv = plsc.load_gather(qnum, [rows, col])
                    kv = plsc.load_gather(kbuf, [rows, col])
                    return acc + qv * kv
                sc = lax.fori_loop(0, DH, _dot, z16)
                eh = jnp.exp(jnp.minimum(sc, 60.0))
                plsc.store_scatter(ex_v, [rows, cbase + h], eh)
        cv = pltpu.async_copy(v_hbm.at[idx_s], kbuf, sem_kv)
        cv.wait()
        for g in range(CHUNK // LANES):
            rows = jnp.arange(LANES, dtype=jnp.int32) + (g * LANES)
            cbase = (idx_d[g * LANES:(g + 1) * LANES] & 7) << 4
            for h in range(H):
                eh = plsc.load_gather(ex_v, [rows, cbase + h])

                def _val(j, _, h=h, rows=rows, eh=eh):
                    colv = jnp.full((LANES,), h * DH, jnp.int32) + j
                    vv = plsc.load_gather(kbuf, [rows, colv])
                    plsc.store_scatter(qnum, [rows, colv], eh * vv)
                    return 0
                lax.fori_loop(0, DH, _val, 0)
        pltpu.sync_copy(qnum, num_sh.at[idx_d], add=True)
        pltpu.sync_copy(ex_v, den_sh.at[idx_d8], add=True)
        for g in range(CHUNK // LANES):
            rows = jnp.arange(LANES, dtype=jnp.int32) + (g * LANES)
            cbase = (idx_d[g * LANES:(g + 1) * LANES] & 7) << 4
            for h in range(H):
                plsc.store_scatter(ex_v, [rows, cbase + h], z16)
        return 0

    lax.fori_loop(0, nchunk, _chunk, 0)
    plsc.subcore_barrier()

    pltpu.sync_copy(num_sh.at[pl.ds(r0, rpt)], num_hbm.at[c, pl.ds(r0, rpt)])
    pltpu.sync_copy(den_sh.at[pl.ds(s * (rpt // 8), rpt // 8)],
                    den_hbm.at[c, pl.ds(s * (rpt // 8), rpt // 8)])


def _edges(q, k, v, dst, src):
    n = q.shape[0]
    e = dst.shape[0]
    assert e % (NC * NS) == 0
    ept = e // (NC * NS)
    assert ept % CHUNK == 0
    npad = -(-n // (NS * 64)) * (NS * 64)
    mesh = plsc.VectorSubcoreMesh(core_axis_name="c", subcore_axis_name="s")
    kfn = pl.kernel(
        functools.partial(_edge_body, npad, ept),
        out_type=[
            jax.ShapeDtypeStruct((NC, npad, D), jnp.float32),
            jax.ShapeDtypeStruct((NC, npad // 8, D), jnp.float32),
        ],
        mesh=mesh,
        compiler_params=pltpu.CompilerParams(needs_layout_passes=False),
        scratch_types=[
            pltpu.VMEM((CHUNK,), jnp.int32),
            pltpu.VMEM((CHUNK,), jnp.int32),
            pltpu.VMEM((CHUNK,), jnp.int32),
            pltpu.VMEM((CHUNK, D), jnp.float32),
            pltpu.VMEM((CHUNK, D), jnp.float32),
            pltpu.VMEM((CHUNK, D), jnp.float32),
            pltpu.VMEM_SHARED((npad, D), jnp.float32),
            pltpu.VMEM_SHARED((npad // 8, D), jnp.float32),
            pltpu.SemaphoreType.DMA,
            pltpu.SemaphoreType.DMA,
        ],
    )
    num, den8 = kfn(q, k, v, dst, src)
    return num, den8.reshape(NC, npad, DEN_W)


def _fin_body(num_ref, den_ref, sel_ref, o_ref):
    d = den_ref[0] + den_ref[1]
    dx = jnp.dot(d, sel_ref[...], preferred_element_type=jnp.float32) + 1e-16
    o_ref[...] = (num_ref[0] + num_ref[1]) / dx


def _finish(num, den, sel):
    n = num.shape[1]
    return pl.pallas_call(
        _fin_body,
        out_shape=jax.ShapeDtypeStruct((n, D), jnp.float32),
    )(num, den, sel)


def kernel(node_states, edge_indices, Wq, bq, Wk, bk, Wv, bv):
    b, n, d = node_states.shape
    x = node_states.reshape(b * n, d)
    bpad = jnp.zeros((8, d), jnp.float32).at[0].set(bq).at[1].set(bk).at[2].set(bv)
    q, k, v = _qkv(x, Wq, Wk, Wv, bpad)
    dst = edge_indices[1]
    src = edge_indices[2]
    num, den = _edges(q, k, v, dst, src)
    sel = jnp.concatenate(
        [jnp.repeat(jnp.eye(H, dtype=jnp.float32), DH, axis=1),
         jnp.zeros((DEN_W - H, D), jnp.float32)], axis=0)
    out = _finish(num, den, sel)
    return out[:n].reshape(b, n, d)
